# Initial kernel scaffold; baseline (speedup 1.0000x reference)
import jax
import jax.numpy as jnp
from jax.experimental import pallas as pl


def kernel(steps, index, global_step):
    raise NotImplementedError("write your pallas kernel here")


# speedup vs baseline: 3.5691x; 3.5691x over previous
import functools

import jax
import jax.numpy as jnp
from jax import lax
from jax.experimental import pallas as pl
from jax.experimental.pallas import tpu as pltpu
from jax.experimental.pallas import tpu_sc as plsc

NUM_SUBCORES = 16
BATCH = 128
GROUP = 8


def _make_scatter_kernel(num_slots: int, num_indices: int):
  assert num_indices % (NUM_SUBCORES * BATCH) == 0
  rows_per_tile = num_indices // (NUM_SUBCORES * BATCH)
  assert rows_per_tile % GROUP == 0
  chunk = (num_slots // NUM_SUBCORES) // 8 * 8
  tail = num_slots - chunk * NUM_SUBCORES
  n_sub = 6
  assert chunk % (n_sub * 8) == 0
  sub = chunk // n_sub

  mesh = plsc.VectorSubcoreMesh(
      core_axis_name="c", subcore_axis_name="s", num_cores=1)

  @functools.partial(
      pl.kernel,
      out_type=jax.ShapeDtypeStruct((num_slots,), jnp.float32),
      mesh=mesh,
      scratch_types=[
          pltpu.VMEM((rows_per_tile, BATCH), jnp.int32),
          pltpu.VMEM((BATCH,), jnp.float32),
          pltpu.VMEM((16,), jnp.float32),
          pltpu.VMEM((sub,), jnp.float32),
          pltpu.VMEM((sub,), jnp.float32),
          pltpu.SemaphoreType.DMA,
          pltpu.SemaphoreType.DMA,
          pltpu.SemaphoreType.DMA,
          pltpu.SemaphoreType.DMA,
      ],
  )
  def scatter_kernel(steps_hbm, idx_hbm, gs_hbm, out_hbm,
                     idx_v, val_v, gs_v, buf0, buf1,
                     idx_sem, buf_sem0, buf_sem1, sem):
    wid = lax.axis_index("s")

    idx_cp = pltpu.async_copy(idx_hbm.at[wid], idx_v, idx_sem)

    pltpu.sync_copy(gs_hbm, gs_v)
    gs_vec = gs_v[...]
    for i in range(BATCH // 16):
      val_v[pl.ds(i * 16, 16)] = gs_vec

    off = wid * chunk
    bufs = (buf0, buf1)
    sems = (buf_sem0, buf_sem1)
    loads = [None, None]
    loads[0] = pltpu.async_copy(steps_hbm.at[pl.ds(off, sub)], buf0, buf_sem0)
    for k in range(n_sub):
      if k + 1 < n_sub:
        loads[(k + 1) % 2] = pltpu.async_copy(
            steps_hbm.at[pl.ds(off + (k + 1) * sub, sub)],
            bufs[(k + 1) % 2], sems[(k + 1) % 2])
      loads[k % 2].wait()
      pltpu.sync_copy(bufs[k % 2], out_hbm.at[pl.ds(off + k * sub, sub)])
    if tail:
      @pl.when(wid == NUM_SUBCORES - 1)
      def _():
        pltpu.sync_copy(steps_hbm.at[pl.ds(chunk * NUM_SUBCORES, tail)],
                        buf0.at[pl.ds(0, tail)])
        pltpu.sync_copy(buf0.at[pl.ds(0, tail)],
                        out_hbm.at[pl.ds(chunk * NUM_SUBCORES, tail)])

    idx_cp.wait()
    plsc.subcore_barrier()

    def group_body(g, _):
      descs = [
          pltpu.async_copy(val_v, out_hbm.at[idx_v.at[g * GROUP + j]], sem)
          for j in range(GROUP)
      ]
      for d in descs:
        d.wait()
      return 0

    lax.fori_loop(0, rows_per_tile // GROUP, group_body, 0)

  return scatter_kernel


def kernel(steps, index, global_step):
  num_slots = steps.shape[0]
  num_indices = index.shape[0]
  rows_per_tile = num_indices // (NUM_SUBCORES * BATCH)
  idx3 = jnp.reshape(index, (NUM_SUBCORES, rows_per_tile, BATCH))
  gs = jnp.full((16,), global_step, dtype=jnp.float32)
  f = _make_scatter_kernel(num_slots, num_indices)
  return f(steps, idx3, gs)
